# baseline (speedup 1.0000x reference)
import jax
import jax.numpy as jnp
from jax.experimental import pallas as pl

B, N, NF, EF = 64, 64, 128, 4
HID, MSG, T, ENN_H, GATH, MLP_H, FADD = 128, 64, 3, 64, 128, 128, 32

_MPNN_KEYS = ('W_embed', 'enn_W1', 'enn_b1', 'enn_W2', 'enn_b2', 'W_msg',
              'gru_Wi', 'gru_Wh', 'gru_bi', 'gru_bh',
              'att_W1', 'att_b1', 'att_W2', 'att_b2',
              'emb_W1', 'emb_b1', 'emb_W2', 'emb_b2')
_PG_KEYS = _MPNN_KEYS + ('mlp1_W1', 'mlp1_b1', 'mlp1_W2', 'mlp1_b2',
                         'mlp2_W1', 'mlp2_b1', 'mlp2_W2', 'mlp2_b2')
_PC_KEYS = _MPNN_KEYS + ('out_W1', 'out_b1', 'out_W2', 'out_b2')


def _mpnn(X, nodes, p):
    h = jnp.dot(nodes, p['W_embed'], preferred_element_type=jnp.float32)
    emask = (jnp.sum(jnp.abs(X), axis=-1, keepdims=True) > 1e-6).astype(jnp.float32)
    a1 = jnp.maximum(jnp.dot(X, p['enn_W1'], preferred_element_type=jnp.float32)
                     + p['enn_b1'], 0.0)
    gate = jnp.dot(a1, p['enn_W2'], preferred_element_type=jnp.float32) + p['enn_b2']
    gm3 = (gate * emask).reshape(N, N, MSG)
    for _ in range(T):
        hj = jnp.dot(h, p['W_msg'], preferred_element_type=jnp.float32)
        m = jnp.sum(gm3 * hj[None, :, :], axis=1)
        gi = jnp.dot(m, p['gru_Wi'], preferred_element_type=jnp.float32) + p['gru_bi']
        gh = jnp.dot(h, p['gru_Wh'], preferred_element_type=jnp.float32) + p['gru_bh']
        iz, ir, inn = gi[:, :HID], gi[:, HID:2 * HID], gi[:, 2 * HID:]
        hz, hr, hn = gh[:, :HID], gh[:, HID:2 * HID], gh[:, 2 * HID:]
        z = jax.nn.sigmoid(iz + hz)
        r = jax.nn.sigmoid(ir + hr)
        nmsg = jnp.tanh(inn + r * hn)
        h = (1.0 - z) * nmsg + z * h
    return h


def _gather(h, nodes, p):
    cat = jnp.concatenate([h, nodes], axis=-1)
    att = jax.nn.sigmoid(
        jnp.dot(jnp.maximum(jnp.dot(cat, p['att_W1'], preferred_element_type=jnp.float32)
                            + p['att_b1'], 0.0),
                p['att_W2'], preferred_element_type=jnp.float32) + p['att_b2'])
    emb = jnp.dot(jnp.maximum(jnp.dot(h, p['emb_W1'], preferred_element_type=jnp.float32)
                              + p['emb_b1'], 0.0),
                  p['emb_W2'], preferred_element_type=jnp.float32) + p['emb_b2']
    return jnp.sum(att * emb, axis=0, keepdims=True)


def _tc_body(ln_ref, le_ref, fn_ref, fe_ref, *refs):
    npg, npc = len(_PG_KEYS), len(_PC_KEYS)
    pg = {k: refs[i][...] for i, k in enumerate(_PG_KEYS)}
    pc = {k: refs[npg + i][...] for i, k in enumerate(_PC_KEYS)}
    ea_ref, ec_ref, et_ref, idx_ref = refs[npg + npc:]

    ln = ln_ref[0]
    fn = fn_ref[0]
    Xl = le_ref[0]
    Xf = fe_ref[0]

    hl = _mpnn(Xl, ln, pg)
    hf = _mpnn(Xf, fn, pg)
    gl = _gather(hl, ln, pg)
    gf = _gather(hf, fn, pg)

    no = jnp.dot(jnp.maximum(jnp.dot(hl, pg['mlp1_W1'], preferred_element_type=jnp.float32)
                             + pg['mlp1_b1'], 0.0),
                 pg['mlp1_W2'], preferred_element_type=jnp.float32) + pg['mlp1_b2']
    na = no[:, :FADD]
    nc = no[:, FADD:FADD + EF]

    cat2 = jnp.concatenate([gl, gf], axis=-1)
    ft = jnp.dot(jnp.maximum(jnp.dot(cat2, pg['mlp2_W1'], preferred_element_type=jnp.float32)
                             + pg['mlp2_b1'], 0.0),
                 pg['mlp2_W2'], preferred_element_type=jnp.float32) + pg['mlp2_b2']

    mx = jnp.maximum(jnp.maximum(jnp.max(na), jnp.max(nc)), ft[0, 0])
    sa = jnp.exp(na - mx)
    sc = jnp.exp(nc - mx)
    st = jnp.exp(ft - mx)
    zsum = jnp.sum(sa) + jnp.sum(sc) + st[0, 0]
    inv = 1.0 / zsum
    ea_ref[0] = sa * inv
    ec_ref[0] = sc * inv
    et_ref[0] = st * inv

    hc = _mpnn(Xl, ln, pc)
    co = jnp.dot(jnp.maximum(jnp.dot(hc, pc['out_W1'], preferred_element_type=jnp.float32)
                             + pc['out_b1'], 0.0),
                 pc['out_W2'], preferred_element_type=jnp.float32) + pc['out_b2']
    iot = jax.lax.broadcasted_iota(jnp.int32, (N, 1), 0)
    m1 = jnp.max(co, axis=0, keepdims=True)
    i1 = jnp.min(jnp.where(co >= m1, iot, N), axis=0, keepdims=True)
    co2 = jnp.where(iot == i1, -jnp.inf, co)
    m2 = jnp.max(co2, axis=0, keepdims=True)
    i2 = jnp.min(jnp.where(co2 >= m2, iot, N), axis=0, keepdims=True)
    idx_ref[0] = jnp.concatenate([i1, i2], axis=1)


def kernel(linker_nodes, linker_edges, fragment_nodes, fragment_edges,
           params_gen, params_con):
    le = linker_edges.reshape(B, N * N, EF)
    fe = fragment_edges.reshape(B, N * N, EF)

    def b2(x):
        return x.reshape(1, -1) if x.ndim == 1 else x

    wg = [b2(params_gen[k]) for k in _PG_KEYS]
    wc = [b2(params_con[k]) for k in _PC_KEYS]

    def bspec(shape):
        nd = len(shape)
        return pl.BlockSpec((1,) + shape[1:], lambda b: (b,) + (0,) * (nd - 1))

    def wspec(x):
        nd = x.ndim
        return pl.BlockSpec(x.shape, lambda b: (0,) * nd)

    in_specs = [bspec((B, N, NF)), bspec((B, N * N, EF)),
                bspec((B, N, NF)), bspec((B, N * N, EF))]
    in_specs += [wspec(x) for x in wg + wc]

    out_shapes = [jax.ShapeDtypeStruct((B, N, FADD), jnp.float32),
                  jax.ShapeDtypeStruct((B, N, EF), jnp.float32),
                  jax.ShapeDtypeStruct((B, 1, 1), jnp.float32),
                  jax.ShapeDtypeStruct((B, 1, 2), jnp.int32)]
    out_specs = [bspec((B, N, FADD)), bspec((B, N, EF)),
                 bspec((B, 1, 1)), bspec((B, 1, 2))]

    ea, ec, et, idx = pl.pallas_call(
        _tc_body,
        grid=(B,),
        in_specs=in_specs,
        out_specs=out_specs,
        out_shape=out_shapes,
    )(linker_nodes, le, fragment_nodes, fe, *wg, *wc)

    apd = jnp.concatenate([ea.reshape(B, N * FADD), ec.reshape(B, N * EF),
                           et.reshape(B, 1)], axis=-1)
    two_idx = idx.reshape(B, 2)
    tanimoto = jnp.array(1.0, dtype=jnp.float32)
    return (apd, tanimoto, two_idx)
